# Initial kernel scaffold; baseline (speedup 1.0000x reference)
import jax
import jax.numpy as jnp
from jax.experimental import pallas as pl


def kernel(x, pos_table):
    raise NotImplementedError("write your pallas kernel here")


# speedup vs baseline: 1.8485x; 1.8485x over previous
import jax
import jax.numpy as jnp
from jax.experimental import pallas as pl

BT = 512


def _bcast_body(table_ref, out_ref):
    out_ref[...] = table_ref[...][None]


def kernel(x, pos_table):
    B, T, D = x.shape
    grid = (T // BT, B)
    out = pl.pallas_call(
        _bcast_body,
        grid=grid,
        in_specs=[pl.BlockSpec((BT, D), lambda t, b: (t, 0))],
        out_specs=pl.BlockSpec((1, BT, D), lambda t, b: (b, t, 0)),
        out_shape=jax.ShapeDtypeStruct((B, T, D), pos_table.dtype),
    )(pos_table)
    return out
